# baseline (speedup 1.0000x reference)
import jax
import jax.numpy as jnp
from jax import lax
from jax.experimental import pallas as pl
from jax.experimental.pallas import tpu as pltpu
from jax.experimental.pallas import tpu_sc as plsc

N = 10000
NP = 10240
E = 320000
C = 128
NEG = 0.2

NW = 32
CHUNK = 64
NCHUNK = 160
EPW = NCHUNK * CHUNK
EP = NW * EPW
NBUF = 4
RPT = NP // 16
BLK = 1024
GRID = NP // BLK


def _mm_body(x_ref, w_ref, asrc_ref, adst_ref, xp_ref, as_ref, ad_ref):
    xp = jnp.dot(x_ref[...], w_ref[...], preferred_element_type=jnp.float32)
    xp_ref[...] = xp
    as_ref[...] = jnp.sum(xp * asrc_ref[...], axis=1)
    ad_ref[...] = jnp.sum(xp * adst_ref[...], axis=1)


def _mm_call(xpad, W, att_src, att_dst):
    return pl.pallas_call(
        _mm_body,
        grid=(GRID,),
        in_specs=[
            pl.BlockSpec((BLK, C), lambda i: (i, 0)),
            pl.BlockSpec((C, C), lambda i: (0, 0)),
            pl.BlockSpec((1, C), lambda i: (0, 0)),
            pl.BlockSpec((1, C), lambda i: (0, 0)),
        ],
        out_specs=[
            pl.BlockSpec((BLK, C), lambda i: (i, 0)),
            pl.BlockSpec((BLK,), lambda i: (i,)),
            pl.BlockSpec((BLK,), lambda i: (i,)),
        ],
        out_shape=[
            jax.ShapeDtypeStruct((NP, C), jnp.float32),
            jax.ShapeDtypeStruct((NP,), jnp.float32),
            jax.ShapeDtypeStruct((NP,), jnp.float32),
        ],
    )(xpad, W, att_src, att_dst)


def _sc_body(xp_hbm, asrc_hbm, adst_hbm, src_hbm, dst_hbm,
             accp_hbm, denp_hbm,
             asrc_t, adst_t, rows3, sb3, db3, dmw, zbuf,
             sgsem, ssem, dsem, acc_sh, den_sh):
    c = lax.axis_index("c")
    s = lax.axis_index("s")
    wid = s * 2 + c

    pltpu.sync_copy(asrc_hbm, asrc_t)
    pltpu.sync_copy(adst_hbm, adst_t)

    zero16 = jnp.zeros((16,), jnp.float32)

    def zrow(i, _):
        for j in range(8):
            rows3[0, i, pl.ds(j * 16, 16)] = zero16
        return 0
    lax.fori_loop(0, CHUNK, zrow, 0)

    def zb(i, _):
        zbuf[pl.ds(i * 16, 16)] = zero16
        return 0
    lax.fori_loop(0, RPT // 16, zb, 0)

    for r in range(RPT // CHUNK):
        pltpu.sync_copy(rows3.at[0],
                        acc_sh.at[pl.ds(s * RPT + r * CHUNK, CHUNK), :])
    pltpu.sync_copy(zbuf, den_sh.at[pl.ds(s * RPT, RPT)])
    plsc.subcore_barrier()

    def rmax(tbl):
        def body(i, m):
            return jnp.maximum(m, tbl[pl.ds(i * 16, 16)])
        m16 = lax.fori_loop(0, NP // 16, body,
                            jnp.full((16,), -jnp.inf, jnp.float32))
        m = m16[0]
        for i in range(1, 16):
            m = jnp.maximum(m, m16[i])
        return m
    M = rmax(asrc_t) + rmax(adst_t)

    ebase = wid * EPW

    def step(i, _):
        gi = jnp.where(i == NCHUNK, 0, i)
        gslot = lax.rem(i, 3)

        @pl.when(i >= 3)
        def _():
            q = lax.rem(i - 3, 3)
            pltpu.make_async_copy(rows3.at[q], acc_sh.at[db3.at[q]],
                                  ssem).wait()
            pltpu.make_async_copy(dmw.at[q], den_sh.at[db3.at[q]],
                                  dsem).wait()

        pltpu.sync_copy(src_hbm.at[pl.ds(ebase + gi * CHUNK, CHUNK)],
                        sb3.at[gslot])
        pltpu.sync_copy(dst_hbm.at[pl.ds(ebase + gi * CHUNK, CHUNK)],
                        db3.at[gslot])
        pltpu.async_copy(xp_hbm.at[sb3.at[gslot]], rows3.at[gslot], sgsem)

        @pl.when(i >= 1)
        def _():
            p = i - 1
            pslot = lax.rem(p, 3)
            pltpu.make_async_copy(xp_hbm.at[sb3.at[pslot]],
                                  rows3.at[pslot], sgsem).wait()

            def sgroup(g, _):
                si = sb3[pslot, pl.ds(g * 16, 16)]
                di = db3[pslot, pl.ds(g * 16, 16)]
                e = (plsc.load_gather(asrc_t, [si])
                     + plsc.load_gather(adst_t, [di]))
                e = jnp.where(e > 0, e, NEG * e)
                wv = jnp.exp(e - M)
                dmw[pslot, pl.ds(g * 16, 16)] = wv
                for rr in range(16):
                    wr = wv[rr]
                    r = g * 16 + rr
                    for j in range(8):
                        rows3[pslot, r, pl.ds(j * 16, 16)] = (
                            rows3[pslot, r, pl.ds(j * 16, 16)] * wr)
                return 0
            lax.fori_loop(0, CHUNK // 16, sgroup, 0)
            pltpu.async_copy(rows3.at[pslot], acc_sh.at[db3.at[pslot]],
                             ssem, add=True)
            pltpu.async_copy(dmw.at[pslot], den_sh.at[db3.at[pslot]],
                             dsem, add=True)
        return 0
    lax.fori_loop(0, NCHUNK + 1, step, 0)

    pltpu.make_async_copy(xp_hbm.at[sb3.at[lax.rem(NCHUNK, 3)]],
                          rows3.at[lax.rem(NCHUNK, 3)], sgsem).wait()
    for q in (NCHUNK - 2, NCHUNK - 1):
        qs = lax.rem(q, 3)
        pltpu.make_async_copy(rows3.at[qs], acc_sh.at[db3.at[qs]],
                              ssem).wait()
        pltpu.make_async_copy(dmw.at[qs], den_sh.at[db3.at[qs]],
                              dsem).wait()

    plsc.subcore_barrier()
    pltpu.sync_copy(acc_sh.at[pl.ds(s * RPT, RPT), :],
                    accp_hbm.at[c, pl.ds(s * RPT, RPT), :])
    pltpu.sync_copy(den_sh.at[pl.ds(s * RPT, RPT)],
                    denp_hbm.at[c, pl.ds(s * RPT, RPT)])


def _sc_call(xp, asrc, adst, srcp, dstp):
    f = pl.kernel(
        _sc_body,
        out_type=(jax.ShapeDtypeStruct((2, NP, C), jnp.float32),
                  jax.ShapeDtypeStruct((2, NP), jnp.float32)),
        mesh=plsc.VectorSubcoreMesh(core_axis_name="c", subcore_axis_name="s"),
        compiler_params=pltpu.CompilerParams(needs_layout_passes=False),
        scratch_types=[
            pltpu.VMEM((NP,), jnp.float32),
            pltpu.VMEM((NP,), jnp.float32),
            pltpu.VMEM((3, CHUNK, C), jnp.float32),
            pltpu.VMEM((3, CHUNK), jnp.int32),
            pltpu.VMEM((3, CHUNK), jnp.int32),
            pltpu.VMEM((3, CHUNK), jnp.float32),
            pltpu.VMEM((RPT,), jnp.float32),
            pltpu.SemaphoreType.DMA,
            pltpu.SemaphoreType.DMA,
            pltpu.SemaphoreType.DMA,
            pltpu.VMEM_SHARED((NP, C), jnp.float32),
            pltpu.VMEM_SHARED((NP,), jnp.float32),
        ],
    )
    return f(xp, asrc, adst, srcp, dstp)


def _comb_body(acc_ref, den_ref, bias_ref, out_ref):
    a = acc_ref[0] + acc_ref[1]
    d = den_ref[0] + den_ref[1]
    out_ref[...] = a / (d + 1e-16)[:, None] + bias_ref[...]


def _comb_call(accp, denp, bias):
    return pl.pallas_call(
        _comb_body,
        grid=(GRID,),
        in_specs=[
            pl.BlockSpec((2, BLK, C), lambda i: (0, i, 0)),
            pl.BlockSpec((2, BLK), lambda i: (0, i)),
            pl.BlockSpec((1, C), lambda i: (0, 0)),
        ],
        out_specs=pl.BlockSpec((BLK, C), lambda i: (i, 0)),
        out_shape=jax.ShapeDtypeStruct((NP, C), jnp.float32),
    )(accp, denp, bias)


def kernel(x, edge_index, W, att_src, att_dst, bias):
    xpad = jnp.pad(x, ((0, NP - N), (0, 0)))
    srcp = jnp.pad(edge_index[0], (0, EP - E),
                   constant_values=NP - 1)
    dstp = jnp.pad(edge_index[1], (0, EP - E),
                   constant_values=NP - 1)
    xp, asrc, adst = _mm_call(xpad, W, att_src.reshape(1, C),
                              att_dst.reshape(1, C))
    accp, denp = _sc_call(xp, asrc, adst, srcp, dstp)
    out = _comb_call(accp, denp, bias.reshape(1, C))
    return out[:N]
